# Initial kernel scaffold; baseline (speedup 1.0000x reference)
import jax
import jax.numpy as jnp
from jax.experimental import pallas as pl


def kernel(batch, n_tens, scalars):
    raise NotImplementedError("write your pallas kernel here")


# speedup vs baseline: 2.2796x; 2.2796x over previous
import functools

import jax
import jax.numpy as jnp
from jax import lax
from jax.experimental import pallas as pl
from jax.experimental.pallas import tpu as pltpu
from jax.experimental.pallas import tpu_sc as plsc

_B = 2
_N_COARSE = 50000
_NUM_TETS = 100000
_D = 128

_TETS_PER_CHUNK = 32
_IDX_PER_CHUNK = 4 * _TETS_PER_CHUNK
_TOTAL_ROWS = _B * _NUM_TETS
_NUM_CHUNKS = _TOTAL_ROWS // _TETS_PER_CHUNK
_NW = 32
_CHUNKS_HI = -(-_NUM_CHUNKS // _NW)
_REM = _NUM_CHUNKS - (_CHUNKS_HI - 1) * _NW


def _splat(vec, lane):
    idx = jnp.full((16, 1), lane, dtype=jnp.int32)
    dnums = lax.GatherDimensionNumbers(
        offset_dims=(), collapsed_slice_dims=(0,), start_index_map=(0,))
    return lax.gather(vec, idx, dnums, slice_sizes=(1,),
                      mode=lax.GatherScatterMode.PROMISE_IN_BOUNDS)


def _sc_interpolate(table, idx2d, w16):
    mesh = plsc.VectorSubcoreMesh(core_axis_name="c", subcore_axis_name="s")

    @functools.partial(
        pl.kernel,
        out_type=jax.ShapeDtypeStruct((_TOTAL_ROWS, _D), jnp.float32),
        mesh=mesh,
        scratch_types=[
            pltpu.VMEM((_IDX_PER_CHUNK,), jnp.int32),
            pltpu.VMEM((_TETS_PER_CHUNK, 16), jnp.float32),
            pltpu.VMEM((_IDX_PER_CHUNK, _D), jnp.float32),
            pltpu.VMEM((_TETS_PER_CHUNK, _D), jnp.float32),
            pltpu.SemaphoreType.DMA,
        ],
    )
    def k(table_hbm, idx_hbm, w_hbm, out_hbm, idx_v, w_v, g_v, o_v, sem):
        wid = lax.axis_index("s") * 2 + lax.axis_index("c")
        n_chunks = jnp.where(wid < _REM, _CHUNKS_HI, _CHUNKS_HI - 1)

        def chunk_body(i, carry):
            cid = wid + _NW * i
            row0 = cid * _TETS_PER_CHUNK
            srow = lax.rem(row0, _NUM_TETS)
            pltpu.sync_copy(idx_hbm.at[cid], idx_v)
            pltpu.sync_copy(w_hbm.at[pl.ds(srow, _TETS_PER_CHUNK)], w_v)
            pltpu.async_copy(table_hbm.at[idx_v], g_v, sem).wait()

            for t in range(_TETS_PER_CHUNK):
                wrow = w_v[t, :]
                w0 = _splat(wrow, 0)
                w1 = _splat(wrow, 1)
                w2 = _splat(wrow, 2)
                w3 = _splat(wrow, 3)
                r = 4 * t
                for dk in range(_D // 16):
                    s = pl.ds(dk * 16, 16)
                    acc = (w0 * g_v[r, s] + w1 * g_v[r + 1, s]
                           + w2 * g_v[r + 2, s] + w3 * g_v[r + 3, s])
                    o_v[t, s] = acc

            pltpu.sync_copy(o_v, out_hbm.at[pl.ds(row0, _TETS_PER_CHUNK)])
            return carry

        lax.fori_loop(0, n_chunks, chunk_body, 0)

    return k(table, idx2d, w16)


def kernel(batch, n_tens, scalars):
    table = batch.reshape(_B * _N_COARSE, _D)
    pad = n_tens == _N_COARSE
    nc = jnp.where(pad, 0, n_tens).astype(jnp.int32)
    idx2d = jnp.concatenate([nc, nc + _N_COARSE]).reshape(_NUM_CHUNKS,
                                                          _IDX_PER_CHUNK)
    w = jnp.where(pad.reshape(_NUM_TETS, 4), 0.0, scalars)
    w16 = jnp.pad(w, ((0, 0), (0, 12)))
    out = _sc_interpolate(table, idx2d, w16)
    return out.reshape(_B, _NUM_TETS, _D)


# speedup vs baseline: 2.3478x; 1.0299x over previous
import functools

import jax
import jax.numpy as jnp
from jax import lax
from jax.experimental import pallas as pl
from jax.experimental.pallas import tpu as pltpu
from jax.experimental.pallas import tpu_sc as plsc

_B = 2
_N_COARSE = 50000
_NUM_TETS = 100000
_D = 128

_T = 64
_G = 128
_NG = _T * 4 // _G
_TOTAL_ROWS = _B * _NUM_TETS
_NCH = _TOTAL_ROWS // _T
_NW = 32
_IT = -(-_NCH // _NW)


def _splat(vec, lane):
    idx = jnp.full((16, 1), lane, dtype=jnp.int32)
    dnums = lax.GatherDimensionNumbers(
        offset_dims=(), collapsed_slice_dims=(0,), start_index_map=(0,))
    return lax.gather(vec, idx, dnums, slice_sizes=(1,),
                      mode=lax.GatherScatterMode.PROMISE_IN_BOUNDS)


def _sc_interpolate(table, idx3d, wrows):
    mesh = plsc.VectorSubcoreMesh(core_axis_name="c", subcore_axis_name="s")

    @functools.partial(
        pl.kernel,
        out_type=jax.ShapeDtypeStruct((_TOTAL_ROWS, _D), jnp.float32),
        mesh=mesh,
        scratch_types=[
            pltpu.VMEM((2, _NG, _G), jnp.int32),
            pltpu.VMEM((2, _T, 16), jnp.float32),
            pltpu.VMEM((2, _NG * _G, _D), jnp.float32),
            pltpu.VMEM((_T, _D), jnp.float32),
            pltpu.SemaphoreType.DMA,
            pltpu.SemaphoreType.DMA,
        ],
    )
    def k(table_hbm, idx_hbm, w_hbm, out_hbm, idx_v, w_v, g_v, o_v, s0, s1):
        wid = lax.axis_index("s") * 2 + lax.axis_index("c")
        sems = [s0, s1]

        def chunk_id(i):
            return jnp.minimum(wid + _NW * i, _NCH - 1)

        def start(i, b):
            cid = chunk_id(i)
            row0 = cid * _T
            pltpu.sync_copy(idx_hbm.at[cid], idx_v.at[b])
            pltpu.sync_copy(w_hbm.at[pl.ds(row0, _T)], w_v.at[b])
            for j in range(_NG):
                pltpu.async_copy(table_hbm.at[idx_v.at[b, j]],
                                 g_v.at[b, pl.ds(j * _G, _G)], sems[b])

        def wait_gathers(b):
            for j in range(_NG):
                pltpu.make_async_copy(table_hbm.at[idx_v.at[b, j]],
                                      g_v.at[b, pl.ds(j * _G, _G)],
                                      sems[b]).wait()

        def compute_store(i, b):
            for t in range(_T):
                wrow = w_v[b, t, :]
                w0 = _splat(wrow, 0)
                w1 = _splat(wrow, 1)
                w2 = _splat(wrow, 2)
                w3 = _splat(wrow, 3)
                r = 4 * t
                for dk in range(_D // 16):
                    s = pl.ds(dk * 16, 16)
                    acc = (w0 * g_v[b, r, s] + w1 * g_v[b, r + 1, s]
                           + w2 * g_v[b, r + 2, s] + w3 * g_v[b, r + 3, s])
                    o_v[t, s] = acc
            row0 = chunk_id(i) * _T
            pltpu.sync_copy(o_v, out_hbm.at[pl.ds(row0, _T)])

        start(0, 0)

        def body(ii, carry):
            for b in range(2):
                i = 2 * ii + b
                start(i + 1, 1 - b)
                wait_gathers(b)
                compute_store(i, b)
            return carry

        lax.fori_loop(0, _IT // 2, body, 0)
        wait_gathers(_IT % 2)

    return k(table, idx3d, wrows)


def kernel(batch, n_tens, scalars):
    table = batch.reshape(_B * _N_COARSE, _D)
    pad = n_tens == _N_COARSE
    nc = jnp.where(pad, 0, n_tens).astype(jnp.int32)
    idx3d = jnp.concatenate([nc, nc + _N_COARSE]).reshape(_NCH, _NG, _G)
    w = jnp.where(pad.reshape(_NUM_TETS, 4), 0.0, scalars)
    w16 = jnp.pad(w, ((0, 0), (0, 12)))
    wrows = jnp.concatenate([w16, w16])
    out = _sc_interpolate(table, idx3d, wrows)
    return out.reshape(_B, _NUM_TETS, _D)
